# baseline (speedup 1.0000x reference)
import functools

import jax
import jax.numpy as jnp
from jax import lax
from jax.experimental import pallas as pl
from jax.experimental.pallas import tpu as pltpu
from jax.experimental.pallas import tpu_sc as plsc

N = 10000
E = 160000
CH = 256
NTILES = 32
CPT = 20
TOT_CH = NTILES * CPT
EPAD = TOT_CH * CH
REAL_CH = E // CH
NS = 16
NPAD = 10240
RPT = NPAD // NS
RB = 2048
EP8 = EPAD // 8
NP8 = NPAD // 8
NEG = -1e30

f32 = jnp.float32


def _fold(params):
    mm = functools.partial(jnp.matmul, precision=jax.lax.Precision.HIGHEST)
    ee, en = params['enc_edge'], params['enc_node']
    ce, cn = params['core_edge'], params['core_node']
    de, dn = params['dec_edge'], params['dec_node']
    on, oe = params['out_node'], params['out_edge']
    f = {}
    f['ee_W1s'] = ee['W1'][:128]
    f['ee_W1d'] = ee['W1'][128:256]
    f['ee_W1e'] = ee['W1'][256:]
    f['ee_b1'] = ee['b1']
    f['en_Ma'] = mm(ee['W2'], en['W1'][:272])
    f['en_ca'] = mm(ee['b2'], en['W1'][:272])
    f['en_W1u'] = en['W1'][272:288]
    f['en_W1v'] = en['W1'][288:]
    f['en_b1'] = en['b1']
    W1 = ce['W1']
    W1s_v0, W1s_v = W1[0:416], W1[416:832]
    W1d_v0, W1d_v = W1[832:1248], W1[1248:1664]
    W1e_e0, W1e_e = W1[1664:1936], W1[1936:2208]
    f['ce_Ms_v0'] = mm(en['W2'], W1s_v0)
    f['ce_Md_v0'] = mm(en['W2'], W1d_v0)
    f['ce_Ae0'] = mm(ee['W2'], W1e_e0)
    f['ce_Ms_v_en'] = mm(en['W2'], W1s_v)
    f['ce_Md_v_en'] = mm(en['W2'], W1d_v)
    f['ce_Ms_v_cn'] = mm(cn['W2'], W1s_v)
    f['ce_Md_v_cn'] = mm(cn['W2'], W1d_v)
    f['ce_Ae_ee'] = mm(ee['W2'], W1e_e)
    f['ce_Ae_ce'] = mm(ce['W2'], W1e_e)
    f['ce_c_v0'] = mm(en['b2'], W1s_v0) + mm(en['b2'], W1d_v0) + mm(ee['b2'], W1e_e0)
    f['ce_c_en'] = mm(en['b2'], W1s_v) + mm(en['b2'], W1d_v) + mm(ee['b2'], W1e_e)
    f['ce_c_cn'] = mm(cn['b2'], W1s_v) + mm(cn['b2'], W1d_v) + mm(ce['b2'], W1e_e)
    f['ce_b1'] = ce['b1']
    W1 = cn['W1']
    f['cn_Ma'] = mm(ce['W2'], W1[:272])
    f['cn_ca'] = mm(ce['b2'], W1[:272])
    W1v_v0, W1v_v = W1[304:720], W1[720:1136]
    f['cn_Mv0'] = mm(en['W2'], W1v_v0)
    f['cn_Mv_en'] = mm(en['W2'], W1v_v)
    f['cn_Mv_cn'] = mm(cn['W2'], W1v_v)
    f['cn_c_v0'] = mm(en['b2'], W1v_v0)
    f['cn_c_en'] = mm(en['b2'], W1v_v)
    f['cn_c_cn'] = mm(cn['b2'], W1v_v)
    f['cn_b1'] = cn['b1']
    W1 = de['W1']
    f['de_Ms'] = mm(cn['W2'], W1[0:416])
    f['de_Md'] = mm(cn['W2'], W1[416:832])
    f['de_Ae'] = mm(ce['W2'], W1[832:1104])
    f['de_c'] = mm(cn['b2'], W1[0:416]) + mm(cn['b2'], W1[416:832]) + mm(ce['b2'], W1[832:1104])
    f['de_b1'] = de['b1']
    W1 = dn['W1']
    f['dn_Ma'] = mm(de['W2'], W1[:11])
    f['dn_ca'] = mm(de['b2'], W1[:11])
    f['dn_Mv'] = mm(cn['W2'], W1[27:443])
    f['dn_cv'] = mm(cn['b2'], W1[27:443])
    f['dn_b1'] = dn['b1']
    f['ov_W'] = mm(dn['W2'], on['W'])
    f['ov_b'] = mm(dn['b2'], on['W']) + on['b']
    f['oe_W'] = mm(de['W2'], oe['W'])
    f['oe_b'] = mm(de['b2'], oe['W']) + oe['b']
    return f


def _bc8(v):
    return jnp.broadcast_to(v[None, :], (8, v.shape[0]))


_HI = jax.lax.Precision.HIGHEST


def _dot(a, b):
    return jax.lax.dot_general(a, b, (((1,), (0,)), ((), ())),
                               precision=_HI)


_EYE8 = None


def _kr(A):
    return jnp.kron(jnp.eye(8, dtype=f32), A)


def _tile8(b):
    return jnp.tile(b, 8)


def _sc_round(Ta, Tb, rowi, coli, colsi, X, zeros_hbm, ones_hbm, with_deg):
    mesh = plsc.VectorSubcoreMesh(core_axis_name="c", subcore_axis_name="s")
    out_type = [
        jax.ShapeDtypeStruct((2 * NPAD, 16), f32),
        jax.ShapeDtypeStruct((EPAD, 16), f32),
    ]
    if with_deg:
        out_type.append(jax.ShapeDtypeStruct((2 * NPAD, 16), f32))
    scratch = [
        pltpu.VMEM((CPT, CH), jnp.int32),
        pltpu.VMEM((CPT, CH), jnp.int32),
        pltpu.VMEM((CPT, CH), jnp.int32),
    ] + [pltpu.VMEM((CH, 16), f32)] * 16 + [
        pltpu.VMEM((CH, 16), f32),
        pltpu.VMEM_SHARED((NPAD, 16), f32),
    ] + [pltpu.SemaphoreType.DMA] * 8
    if with_deg:
        scratch.append(pltpu.VMEM_SHARED((NPAD, 16), f32))

    @functools.partial(
        pl.kernel, mesh=mesh, out_type=out_type, scratch_types=scratch,
        compiler_params=pltpu.CompilerParams(use_tc_tiling_on_sc=False))
    def k(ta, tb, ri, ci, cs, x, zz, oo, s_out, h_out, *rest):
        if with_deg:
            d_out = rest[0]
            acc2 = rest[-1]
            rest = rest[1:-1]
        riv, civ, csv = rest[0:3]
        abuf = rest[3:7]
        bbuf = rest[7:11]
        xbuf = rest[11:15]
        hbuf = rest[15:19]
        onev, acc = rest[19:21]
        sems = rest[21:25]
        semw = rest[25:29]
        c = lax.axis_index("c")
        s = lax.axis_index("s")
        base_ch = c * (NS * CPT) + s * CPT

        pltpu.sync_copy(zz.at[pl.ds(s * RPT, RPT)], acc.at[pl.ds(s * RPT, RPT)])
        if with_deg:
            pltpu.sync_copy(zz.at[pl.ds(s * RPT, RPT)],
                            acc2.at[pl.ds(s * RPT, RPT)])
        pltpu.sync_copy(oo, onev)
        pltpu.sync_copy(ri.at[pl.ds(base_ch, CPT)], riv)
        pltpu.sync_copy(ci.at[pl.ds(base_ch, CPT)], civ)
        pltpu.sync_copy(cs.at[pl.ds(base_ch, CPT)], csv)
        plsc.subcore_barrier()

        def issue(j, b):
            ach = base_ch + j
            pltpu.async_copy(ta.at[riv.at[j]], abuf[b], sems[b])
            pltpu.async_copy(tb.at[civ.at[j]], bbuf[b], sems[b])
            pltpu.async_copy(x.at[pl.ds(ach * CH, CH)], xbuf[b], sems[b])

        for jj in range(4):
            issue(jj, jj)

        def step(i, carry):
            for b in range(4):
                j = 4 * i + b
                ach = base_ch + j
                pltpu.make_async_copy(ta.at[riv.at[j]], abuf[b], sems[b]).wait()
                pltpu.make_async_copy(tb.at[civ.at[j]], bbuf[b], sems[b]).wait()
                pltpu.make_async_copy(x.at[pl.ds(ach * CH, CH)], xbuf[b],
                                      sems[b]).wait()

                @pl.when(j >= 4)
                def _(j=j, b=b):
                    pltpu.make_async_copy(
                        hbuf[b], h_out.at[pl.ds((ach - 4) * CH, CH)],
                        semw[b]).wait()

                def inner(t, _, b=b):
                    base = t * 8
                    for u in range(8):
                        e = base + u
                        v = abuf[b][e] + bbuf[b][e] + xbuf[b][e]
                        hbuf[b][e] = jnp.maximum(v, 0.0)
                    return 0

                lax.fori_loop(0, CH // 8, inner, 0, unroll=False)
                pltpu.sync_copy(hbuf[b], acc.at[csv.at[j]], add=True)
                if with_deg:
                    pltpu.sync_copy(onev, acc2.at[csv.at[j]], add=True)
                pltpu.async_copy(hbuf[b], h_out.at[pl.ds(ach * CH, CH)],
                                 semw[b])

                @pl.when(j + 4 < CPT)
                def _(j=j, b=b):
                    issue(j + 4, b)
            return carry

        lax.fori_loop(0, CPT // 4, step, 0, unroll=False)
        for b in range(4):
            j = CPT - 4 + b
            ach = base_ch + j
            pltpu.make_async_copy(hbuf[b], h_out.at[pl.ds(ach * CH, CH)],
                                  semw[b]).wait()
        plsc.subcore_barrier()
        pltpu.sync_copy(acc.at[pl.ds(s * RPT, RPT)],
                        s_out.at[pl.ds(c * NPAD + s * RPT, RPT)])
        if with_deg:
            pltpu.sync_copy(acc2.at[pl.ds(s * RPT, RPT)],
                            d_out.at[pl.ds(c * NPAD + s * RPT, RPT)])

    res = k(Ta, Tb, rowi, coli, colsi, X.reshape(EPAD, 16), zeros_hbm,
            ones_hbm)
    return (res[0], res[1].reshape(EP8, 128)) + tuple(res[2:])


def _tc_node(S, D, terms, adds, Ma, ca, bias, out_specs):
    n_t = len(terms)
    n_a = len(adds)
    n_o = len(out_specs)

    def body(*refs):
        s_ref, d_ref, ma_ref, ca_ref, b_ref = refs[:5]
        arr_refs = refs[5:5 + n_t]
        mat_refs = refs[5 + n_t:5 + 2 * n_t]
        add_refs = refs[5 + 2 * n_t:5 + 2 * n_t + n_a]
        omat_refs = refs[5 + 2 * n_t + n_a:5 + 2 * n_t + n_a + n_o]
        obias_refs = refs[5 + 2 * n_t + n_a + n_o:5 + 2 * n_t + n_a + 2 * n_o]
        outs = refs[5 + 2 * n_t + n_a + 2 * n_o:]
        ssum = s_ref[0] + s_ref[1]
        deg = d_ref[0] + d_ref[1]
        acc = _dot(ssum, ma_ref[...])
        acc = acc + deg * ca_ref[0:1, :] + b_ref[0:1, :]
        for a, m in zip(arr_refs, mat_refs):
            acc = acc + _dot(a[...], m[...])
        for a in add_refs:
            acc = acc + a[...]
        g = jnp.maximum(acc, 0.0)
        outs[0][...] = g
        for o, m, bb in zip(outs[1:], omat_refs, obias_refs):
            o[...] = _dot(g, m[...]) + bb[0:1, :]

    S3 = S.reshape(2, NP8, 128)
    D3 = D.reshape(2, NP8, 128)
    out_shape = [jax.ShapeDtypeStruct((NP8, 128), f32)] + [
        jax.ShapeDtypeStruct((NP8, 8 * m.shape[1]), f32) for m, _ in out_specs]
    args = [S3, D3, _kr(Ma), _bc8(_tile8(ca)), _bc8(_tile8(bias))]
    args += list(terms_arr := [a for a, _ in terms])
    args += [_kr(m) for _, m in terms]
    args += list(adds)
    args += [_kr(m) for m, _ in out_specs] + [_bc8(_tile8(b))
                                             for _, b in out_specs]
    NB = 640
    sb = pl.BlockSpec((2, NB, 128), lambda i: (0, i, 0))
    nb = pl.BlockSpec((NB, 128), lambda i: (i, 0))
    wb = pl.BlockSpec((128, 128), lambda i: (0, 0))
    cb = pl.BlockSpec((8, 128), lambda i: (0, 0))
    in_specs = [sb, sb, wb, cb, cb] + [nb] * n_t + [wb] * n_t + [nb] * n_a
    in_specs += [pl.BlockSpec((128, 8 * m.shape[1]), lambda i: (0, 0))
                 for m, _ in out_specs]
    in_specs += [pl.BlockSpec((8, 8 * m.shape[1]), lambda i: (0, 0))
                 for m, _ in out_specs]
    o_specs = [nb] + [pl.BlockSpec((NB, 8 * m.shape[1]), lambda i: (i, 0))
                      for m, _ in out_specs]
    return pl.pallas_call(body, grid=(NP8 // NB,), in_specs=in_specs,
                          out_specs=o_specs, out_shape=out_shape)(*args)


def _tc_edge(terms, bias, oute=None):
    n_t = len(terms)
    has_x = n_t > 0
    grid = EP8 // RB

    def body(*refs):
        arr_refs = refs[:n_t]
        mat_refs = refs[n_t:2 * n_t]
        i = 2 * n_t
        if has_x:
            b_ref = refs[i]; i += 1
        if oute is not None:
            he_ref = refs[i]; w_ref = refs[i + 1]; ob_ref = refs[i + 2]; i += 3
        outs = refs[i:]
        oi = 0
        if has_x:
            acc = jnp.zeros((RB, 128), f32) + b_ref[0:1, :]
            for a, m in zip(arr_refs, mat_refs):
                acc = acc + _dot(a[...], m[...])
            outs[oi][...] = acc
            oi += 1
        if oute is not None:
            outs[oi][...] = _dot(he_ref[...], w_ref[...]) + ob_ref[0:1, :]

    eb = pl.BlockSpec((RB, 128), lambda i: (i, 0))
    wb = pl.BlockSpec((128, 128), lambda i: (0, 0))
    in_specs = [eb] * n_t + [wb] * n_t
    args = [a for a, _ in terms] + [_kr(m) for _, m in terms]
    out_shape = []
    out_specs = []
    if has_x:
        in_specs += [pl.BlockSpec((8, 128), lambda i: (0, 0))]
        args += [_bc8(_tile8(bias))]
        out_shape.append(jax.ShapeDtypeStruct((EP8, 128), f32))
        out_specs.append(eb)
    if oute is not None:
        he, w, ob = oute
        in_specs += [eb, pl.BlockSpec((128, 8), lambda i: (0, 0)),
                     pl.BlockSpec((8, 8), lambda i: (0, 0))]
        args += [he, _kr(w), _bc8(_tile8(ob))]
        out_shape.append(jax.ShapeDtypeStruct((EP8, 8), f32))
        out_specs.append(pl.BlockSpec((RB, 8), lambda i: (i, 0)))
    res = pl.pallas_call(body, grid=(grid,), in_specs=in_specs,
                         out_specs=out_specs, out_shape=out_shape)(*args)
    return res


def _tc_prep(na2, u128, f):
    def body(na, uu, ws, wd, wu, wv, b1, ta, tb, be):
        ta[...] = _dot(na[...], ws[...])
        tb[...] = _dot(na[...], wd[...])
        u = _dot(uu[0:1, :], wu[...])
        be[...] = _dot(na[...], wv[...]) + u + b1[0:1, :]

    NB = 640
    nb = pl.BlockSpec((NB, 1024), lambda i: (i, 0))
    ob = pl.BlockSpec((NB, 128), lambda i: (i, 0))
    cb = pl.BlockSpec((8, 128), lambda i: (0, 0))
    wb = pl.BlockSpec((1024, 128), lambda i: (0, 0))
    out_shape = [jax.ShapeDtypeStruct((NP8, 128), f32)] * 3
    return pl.pallas_call(
        body, grid=(NP8 // NB,),
        in_specs=[nb, cb, wb, wb, pl.BlockSpec((128, 128), lambda i: (0, 0)),
                  wb, cb],
        out_specs=[ob, ob, ob], out_shape=out_shape)(
            na2, u128, _kr(f['ee_W1s']), _kr(f['ee_W1d']), _kr(f['en_W1u']),
            _kr(f['en_W1v']), _bc8(_tile8(f['en_b1'])))


def _tc_q1(eap_packed, W1e, b1):
    def body(ea, w, b, out):
        out[...] = _dot(ea[...], w[...]) + b[0:1, :]

    eb = pl.BlockSpec((RB, 128), lambda i: (i, 0))
    return pl.pallas_call(
        body, grid=(EP8 // RB,),
        in_specs=[eb, pl.BlockSpec((128, 128), lambda i: (0, 0)),
                  pl.BlockSpec((8, 128), lambda i: (0, 0))],
        out_specs=eb,
        out_shape=jax.ShapeDtypeStruct((EP8, 128), f32))(
            eap_packed, _kr(W1e), _bc8(_tile8(b1)))


def kernel(node_attr, edge_attr, global_attr, params, edge_index, steps):
    f = _fold(params)
    row = edge_index[0].astype(jnp.int32)
    col = edge_index[1].astype(jnp.int32)
    pad_idx = (jnp.arange(EPAD - E, dtype=jnp.int32) % 64)
    rowp = jnp.concatenate([row, pad_idx]).reshape(TOT_CH, CH)
    colp = jnp.concatenate([col, pad_idx]).reshape(TOT_CH, CH)
    dump_idx = N + (jnp.arange(EPAD - E, dtype=jnp.int32) % (NPAD - N))
    colsp = jnp.concatenate([col, dump_idx]).reshape(TOT_CH, CH)
    zeros_hbm = jnp.zeros((NPAD, 16), f32)
    ones_hbm = jnp.ones((CH, 16), f32)
    eap = jnp.concatenate([edge_attr.reshape(E // 8, 128),
                           jnp.zeros((EP8 - E // 8, 128), f32)])
    na2 = jnp.concatenate([node_attr, jnp.zeros((NPAD - N, 128), f32)]
                          ).reshape(NP8, 1024)
    u128 = jnp.broadcast_to(jnp.tile(global_attr.reshape(16), 8)[None, :],
                            (8, 128))

    Ta0, Tb0, base_en = _tc_prep(na2, u128, f)
    X0 = _tc_q1(eap, f['ee_W1e'], f['ee_b1'])

    S1, h1, D = _sc_round(Ta0.reshape(NPAD, 16), Tb0.reshape(NPAD, 16), rowp,
                         colp, colsp, X0, zeros_hbm, ones_hbm, True)
    g1, TaC1, TbC1 = _tc_node(
        S1, D, [], [base_en], f['en_Ma'], f['en_ca'],
        jnp.zeros(16, f32),
        [(f['ce_Ms_v0'] + f['ce_Ms_v_en'], jnp.zeros(16, f32)),
         (f['ce_Md_v0'] + f['ce_Md_v_en'], jnp.zeros(16, f32))])
    XC1 = _tc_edge([(h1, f['ce_Ae0'] + f['ce_Ae_ee'])],
                   f['ce_c_v0'] + f['ce_c_en'] + f['ce_b1'])[0]

    Sc1, hc1 = _sc_round(TaC1.reshape(NPAD, 16), TbC1.reshape(NPAD, 16),
                         rowp, colp, colsp, XC1, zeros_hbm,
                         ones_hbm, False)
    gc1, TaD1, TbD1 = _tc_node(
        Sc1, D, [(g1, f['cn_Mv0'] + f['cn_Mv_en'])], [], f['cn_Ma'],
        f['cn_ca'],
        f['cn_c_v0'] + f['cn_c_en'] + f['cn_b1'],
        [(f['de_Ms'], jnp.zeros(16, f32)), (f['de_Md'], jnp.zeros(16, f32))])
    XD1 = _tc_edge([(hc1, f['de_Ae'])], f['de_c'] + f['de_b1'])[0]

    Sd1, hd1 = _sc_round(TaD1.reshape(NPAD, 16), TbD1.reshape(NPAD, 16),
                         rowp, colp, colsp, XD1, zeros_hbm,
                         ones_hbm, False)
    gd1, outv1 = _tc_node(
        Sd1, D, [(gc1, f['dn_Mv'])], [], f['dn_Ma'], f['dn_ca'],
        f['dn_cv'] + f['dn_b1'],
        [(f['ov_W'], f['ov_b'])])
    XC2, oute1 = _tc_edge(
        [(h1, f['ce_Ae0']), (hc1, f['ce_Ae_ce'])],
        f['ce_c_v0'] + f['ce_c_cn'] + f['ce_b1'],
        oute=(hd1, f['oe_W'], f['oe_b']))
    TaC2, TbC2 = _tc_tables2(g1, gc1, f['ce_Ms_v0'], f['ce_Ms_v_cn'],
                             f['ce_Md_v0'], f['ce_Md_v_cn'])

    Sc2, hc2 = _sc_round(TaC2.reshape(NPAD, 16), TbC2.reshape(NPAD, 16),
                         rowp, colp, colsp, XC2, zeros_hbm,
                         ones_hbm, False)
    gc2, TaD2, TbD2 = _tc_node(
        Sc2, D, [(g1, f['cn_Mv0']), (gc1, f['cn_Mv_cn'])], [], f['cn_Ma'],
        f['cn_ca'], f['cn_c_v0'] + f['cn_c_cn'] + f['cn_b1'],
        [(f['de_Ms'], jnp.zeros(16, f32)), (f['de_Md'], jnp.zeros(16, f32))])
    XD2 = _tc_edge([(hc2, f['de_Ae'])], f['de_c'] + f['de_b1'])[0]

    Sd2, hd2 = _sc_round(TaD2.reshape(NPAD, 16), TbD2.reshape(NPAD, 16),
                         rowp, colp, colsp, XD2, zeros_hbm,
                         ones_hbm, False)
    gd2, outv2 = _tc_node(
        Sd2, D, [(gc2, f['dn_Mv'])], [], f['dn_Ma'], f['dn_ca'],
        f['dn_cv'] + f['dn_b1'],
        [(f['ov_W'], f['ov_b'])])
    oute2 = _tc_edge([], None, oute=(hd2, f['oe_W'], f['oe_b']))[0]

    outs_v = jnp.stack([outv1.reshape(NPAD, 5)[:N],
                        outv2.reshape(NPAD, 5)[:N]])
    outs_e = jnp.stack([oute1.reshape(EPAD)[:E].reshape(E, 1),
                        oute2.reshape(EPAD)[:E].reshape(E, 1)])
    outs_u = jnp.zeros((2, 1, 16), f32)
    return (outs_v, outs_e, outs_u)


def _tc_tables2(ga, gb, Msa, Msb, Mda, Mdb):
    def body(a, b, m1, m2, m3, m4, ta, tb):
        ta[...] = _dot(a[...], m1[...]) + _dot(b[...], m2[...])
        tb[...] = _dot(a[...], m3[...]) + _dot(b[...], m4[...])

    NB = 640
    nb = pl.BlockSpec((NB, 128), lambda i: (i, 0))
    wb = pl.BlockSpec((128, 128), lambda i: (0, 0))
    out_shape = [jax.ShapeDtypeStruct((NP8, 128), f32)] * 2
    return pl.pallas_call(body, grid=(NP8 // NB,),
                          in_specs=[nb, nb, wb, wb, wb, wb],
                          out_specs=[nb, nb], out_shape=out_shape)(
                              ga, gb, _kr(Msa), _kr(Msb), _kr(Mda), _kr(Mdb))


# speedup vs baseline: 1.0220x; 1.0220x over previous
import functools

import jax
import jax.numpy as jnp
from jax import lax
from jax.experimental import pallas as pl
from jax.experimental.pallas import tpu as pltpu
from jax.experimental.pallas import tpu_sc as plsc

N = 10000
E = 160000
CH = 128
NTILES = 32
CPT = 40
TOT_CH = NTILES * CPT
EPAD = TOT_CH * CH
REAL_CH = E // CH
NS = 16
NPAD = 10240
RPT = NPAD // NS
RB = 2048
EP8 = EPAD // 8
NP8 = NPAD // 8
NEG = -1e30

f32 = jnp.float32


def _fold(params):
    mm = functools.partial(jnp.matmul, precision=jax.lax.Precision.HIGHEST)
    ee, en = params['enc_edge'], params['enc_node']
    ce, cn = params['core_edge'], params['core_node']
    de, dn = params['dec_edge'], params['dec_node']
    on, oe = params['out_node'], params['out_edge']
    f = {}
    f['ee_W1s'] = ee['W1'][:128]
    f['ee_W1d'] = ee['W1'][128:256]
    f['ee_W1e'] = ee['W1'][256:]
    f['ee_b1'] = ee['b1']
    f['en_Ma'] = mm(ee['W2'], en['W1'][:272])
    f['en_ca'] = mm(ee['b2'], en['W1'][:272])
    f['en_W1u'] = en['W1'][272:288]
    f['en_W1v'] = en['W1'][288:]
    f['en_b1'] = en['b1']
    W1 = ce['W1']
    W1s_v0, W1s_v = W1[0:416], W1[416:832]
    W1d_v0, W1d_v = W1[832:1248], W1[1248:1664]
    W1e_e0, W1e_e = W1[1664:1936], W1[1936:2208]
    f['ce_Ms_v0'] = mm(en['W2'], W1s_v0)
    f['ce_Md_v0'] = mm(en['W2'], W1d_v0)
    f['ce_Ae0'] = mm(ee['W2'], W1e_e0)
    f['ce_Ms_v_en'] = mm(en['W2'], W1s_v)
    f['ce_Md_v_en'] = mm(en['W2'], W1d_v)
    f['ce_Ms_v_cn'] = mm(cn['W2'], W1s_v)
    f['ce_Md_v_cn'] = mm(cn['W2'], W1d_v)
    f['ce_Ae_ee'] = mm(ee['W2'], W1e_e)
    f['ce_Ae_ce'] = mm(ce['W2'], W1e_e)
    f['ce_c_v0'] = mm(en['b2'], W1s_v0) + mm(en['b2'], W1d_v0) + mm(ee['b2'], W1e_e0)
    f['ce_c_en'] = mm(en['b2'], W1s_v) + mm(en['b2'], W1d_v) + mm(ee['b2'], W1e_e)
    f['ce_c_cn'] = mm(cn['b2'], W1s_v) + mm(cn['b2'], W1d_v) + mm(ce['b2'], W1e_e)
    f['ce_b1'] = ce['b1']
    W1 = cn['W1']
    f['cn_Ma'] = mm(ce['W2'], W1[:272])
    f['cn_ca'] = mm(ce['b2'], W1[:272])
    W1v_v0, W1v_v = W1[304:720], W1[720:1136]
    f['cn_Mv0'] = mm(en['W2'], W1v_v0)
    f['cn_Mv_en'] = mm(en['W2'], W1v_v)
    f['cn_Mv_cn'] = mm(cn['W2'], W1v_v)
    f['cn_c_v0'] = mm(en['b2'], W1v_v0)
    f['cn_c_en'] = mm(en['b2'], W1v_v)
    f['cn_c_cn'] = mm(cn['b2'], W1v_v)
    f['cn_b1'] = cn['b1']
    W1 = de['W1']
    f['de_Ms'] = mm(cn['W2'], W1[0:416])
    f['de_Md'] = mm(cn['W2'], W1[416:832])
    f['de_Ae'] = mm(ce['W2'], W1[832:1104])
    f['de_c'] = mm(cn['b2'], W1[0:416]) + mm(cn['b2'], W1[416:832]) + mm(ce['b2'], W1[832:1104])
    f['de_b1'] = de['b1']
    W1 = dn['W1']
    f['dn_Ma'] = mm(de['W2'], W1[:11])
    f['dn_ca'] = mm(de['b2'], W1[:11])
    f['dn_Mv'] = mm(cn['W2'], W1[27:443])
    f['dn_cv'] = mm(cn['b2'], W1[27:443])
    f['dn_b1'] = dn['b1']
    f['ov_W'] = mm(dn['W2'], on['W'])
    f['ov_b'] = mm(dn['b2'], on['W']) + on['b']
    f['oe_W'] = mm(de['W2'], oe['W'])
    f['oe_b'] = mm(de['b2'], oe['W']) + oe['b']
    return f


def _bc8(v):
    return jnp.broadcast_to(v[None, :], (8, v.shape[0]))


_HI = jax.lax.Precision.HIGHEST


def _dot(a, b):
    return jax.lax.dot_general(a, b, (((1,), (0,)), ((), ())),
                               precision=_HI)


_EYE8 = None


def _kr(A):
    return jnp.kron(jnp.eye(8, dtype=f32), A)


def _tile8(b):
    return jnp.tile(b, 8)


def _sc_round(Ta, Tb, rowi, coli, colsi, X, zeros_hbm, ones_hbm, with_deg):
    mesh = plsc.VectorSubcoreMesh(core_axis_name="c", subcore_axis_name="s")
    out_type = [
        jax.ShapeDtypeStruct((2 * NPAD, 16), f32),
        jax.ShapeDtypeStruct((EPAD, 16), f32),
    ]
    if with_deg:
        out_type.append(jax.ShapeDtypeStruct((2 * NPAD, 16), f32))
    scratch = [
        pltpu.VMEM((CPT, CH), jnp.int32),
        pltpu.VMEM((CPT, CH), jnp.int32),
        pltpu.VMEM((CPT, CH), jnp.int32),
    ] + [pltpu.VMEM((CH, 16), f32)] * 16 + [
        pltpu.VMEM((CH, 16), f32),
        pltpu.VMEM_SHARED((NPAD, 16), f32),
    ] + [pltpu.SemaphoreType.DMA] * 8
    if with_deg:
        scratch.append(pltpu.VMEM_SHARED((NPAD, 16), f32))

    @functools.partial(
        pl.kernel, mesh=mesh, out_type=out_type, scratch_types=scratch,
        compiler_params=pltpu.CompilerParams(use_tc_tiling_on_sc=False))
    def k(ta, tb, ri, ci, cs, x, zz, oo, s_out, h_out, *rest):
        if with_deg:
            d_out = rest[0]
            acc2 = rest[-1]
            rest = rest[1:-1]
        riv, civ, csv = rest[0:3]
        abuf = rest[3:7]
        bbuf = rest[7:11]
        xbuf = rest[11:15]
        hbuf = rest[15:19]
        onev, acc = rest[19:21]
        sems = rest[21:25]
        semw = rest[25:29]
        c = lax.axis_index("c")
        s = lax.axis_index("s")
        base_ch = c * (NS * CPT) + s * CPT

        pltpu.sync_copy(zz.at[pl.ds(s * RPT, RPT)], acc.at[pl.ds(s * RPT, RPT)])
        if with_deg:
            pltpu.sync_copy(zz.at[pl.ds(s * RPT, RPT)],
                            acc2.at[pl.ds(s * RPT, RPT)])
        pltpu.sync_copy(oo, onev)
        pltpu.sync_copy(ri.at[pl.ds(base_ch, CPT)], riv)
        pltpu.sync_copy(ci.at[pl.ds(base_ch, CPT)], civ)
        pltpu.sync_copy(cs.at[pl.ds(base_ch, CPT)], csv)
        plsc.subcore_barrier()

        def issue(j, b):
            ach = base_ch + j
            pltpu.async_copy(ta.at[riv.at[j]], abuf[b], sems[b])
            pltpu.async_copy(tb.at[civ.at[j]], bbuf[b], sems[b])
            pltpu.async_copy(x.at[pl.ds(ach * CH, CH)], xbuf[b], sems[b])

        for jj in range(4):
            issue(jj, jj)

        def step(i, carry):
            for b in range(4):
                j = 4 * i + b
                ach = base_ch + j
                pltpu.make_async_copy(ta.at[riv.at[j]], abuf[b], sems[b]).wait()
                pltpu.make_async_copy(tb.at[civ.at[j]], bbuf[b], sems[b]).wait()
                pltpu.make_async_copy(x.at[pl.ds(ach * CH, CH)], xbuf[b],
                                      sems[b]).wait()

                @pl.when(j >= 4)
                def _(j=j, b=b):
                    pltpu.make_async_copy(
                        hbuf[b], h_out.at[pl.ds((ach - 4) * CH, CH)],
                        semw[b]).wait()

                def inner(t, _, b=b):
                    base = t * 8
                    for u in range(8):
                        e = base + u
                        v = abuf[b][e] + bbuf[b][e] + xbuf[b][e]
                        hbuf[b][e] = jnp.maximum(v, 0.0)
                    return 0

                lax.fori_loop(0, CH // 8, inner, 0, unroll=False)
                pltpu.sync_copy(hbuf[b], acc.at[csv.at[j]], add=True)
                if with_deg:
                    pltpu.sync_copy(onev, acc2.at[csv.at[j]], add=True)
                pltpu.async_copy(hbuf[b], h_out.at[pl.ds(ach * CH, CH)],
                                 semw[b])

                @pl.when(j + 4 < CPT)
                def _(j=j, b=b):
                    issue(j + 4, b)
            return carry

        lax.fori_loop(0, CPT // 4, step, 0, unroll=False)
        for b in range(4):
            j = CPT - 4 + b
            ach = base_ch + j
            pltpu.make_async_copy(hbuf[b], h_out.at[pl.ds(ach * CH, CH)],
                                  semw[b]).wait()
        plsc.subcore_barrier()
        pltpu.sync_copy(acc.at[pl.ds(s * RPT, RPT)],
                        s_out.at[pl.ds(c * NPAD + s * RPT, RPT)])
        if with_deg:
            pltpu.sync_copy(acc2.at[pl.ds(s * RPT, RPT)],
                            d_out.at[pl.ds(c * NPAD + s * RPT, RPT)])

    res = k(Ta, Tb, rowi, coli, colsi, X.reshape(EPAD, 16), zeros_hbm,
            ones_hbm)
    return (res[0], res[1].reshape(EP8, 128)) + tuple(res[2:])


def _tc_node(S, D, terms, adds, Ma, ca, bias, out_specs):
    n_t = len(terms)
    n_a = len(adds)
    n_o = len(out_specs)

    def body(*refs):
        s_ref, d_ref, ma_ref, ca_ref, b_ref = refs[:5]
        arr_refs = refs[5:5 + n_t]
        mat_refs = refs[5 + n_t:5 + 2 * n_t]
        add_refs = refs[5 + 2 * n_t:5 + 2 * n_t + n_a]
        omat_refs = refs[5 + 2 * n_t + n_a:5 + 2 * n_t + n_a + n_o]
        obias_refs = refs[5 + 2 * n_t + n_a + n_o:5 + 2 * n_t + n_a + 2 * n_o]
        outs = refs[5 + 2 * n_t + n_a + 2 * n_o:]
        ssum = s_ref[0] + s_ref[1]
        deg = d_ref[0] + d_ref[1]
        acc = _dot(ssum, ma_ref[...])
        acc = acc + deg * ca_ref[0:1, :] + b_ref[0:1, :]
        for a, m in zip(arr_refs, mat_refs):
            acc = acc + _dot(a[...], m[...])
        for a in add_refs:
            acc = acc + a[...]
        g = jnp.maximum(acc, 0.0)
        outs[0][...] = g
        for o, m, bb in zip(outs[1:], omat_refs, obias_refs):
            o[...] = _dot(g, m[...]) + bb[0:1, :]

    S3 = S.reshape(2, NP8, 128)
    D3 = D.reshape(2, NP8, 128)
    out_shape = [jax.ShapeDtypeStruct((NP8, 128), f32)] + [
        jax.ShapeDtypeStruct((NP8, 8 * m.shape[1]), f32) for m, _ in out_specs]
    args = [S3, D3, _kr(Ma), _bc8(_tile8(ca)), _bc8(_tile8(bias))]
    args += list(terms_arr := [a for a, _ in terms])
    args += [_kr(m) for _, m in terms]
    args += list(adds)
    args += [_kr(m) for m, _ in out_specs] + [_bc8(_tile8(b))
                                             for _, b in out_specs]
    NB = 640
    sb = pl.BlockSpec((2, NB, 128), lambda i: (0, i, 0))
    nb = pl.BlockSpec((NB, 128), lambda i: (i, 0))
    wb = pl.BlockSpec((128, 128), lambda i: (0, 0))
    cb = pl.BlockSpec((8, 128), lambda i: (0, 0))
    in_specs = [sb, sb, wb, cb, cb] + [nb] * n_t + [wb] * n_t + [nb] * n_a
    in_specs += [pl.BlockSpec((128, 8 * m.shape[1]), lambda i: (0, 0))
                 for m, _ in out_specs]
    in_specs += [pl.BlockSpec((8, 8 * m.shape[1]), lambda i: (0, 0))
                 for m, _ in out_specs]
    o_specs = [nb] + [pl.BlockSpec((NB, 8 * m.shape[1]), lambda i: (i, 0))
                      for m, _ in out_specs]
    return pl.pallas_call(body, grid=(NP8 // NB,), in_specs=in_specs,
                          out_specs=o_specs, out_shape=out_shape)(*args)


def _tc_edge(terms, bias, oute=None, node=None):
    n_t = len(terms)
    has_x = n_t > 0
    grid = EP8 // RB
    NBn = 640
    if node is not None:
        nn_t = len(node['terms'])
        nn_a = len(node['adds'])
        nn_o = len(node['outs'])

    def body(*refs):
        i = pl.program_id(0)
        arr_refs = refs[:n_t]
        mat_refs = refs[n_t:2 * n_t]
        k = 2 * n_t
        if has_x:
            b_ref = refs[k]; k += 1
        if oute is not None:
            he_ref = refs[k]; w_ref = refs[k + 1]; ob_ref = refs[k + 2]; k += 3
        if node is not None:
            ns_ref = refs[k]; nd_ref = refs[k + 1]
            nma_ref = refs[k + 2]; nca_ref = refs[k + 3]; nb_ref = refs[k + 4]
            k += 5
            nt_arr = refs[k:k + nn_t]; k += nn_t
            nt_mat = refs[k:k + nn_t]; k += nn_t
            na_arr = refs[k:k + nn_a]; k += nn_a
            no_mat = refs[k:k + nn_o]; k += nn_o
            no_b = refs[k:k + nn_o]; k += nn_o
        outs = refs[k:]
        oi = 0
        if has_x:
            acc = jnp.zeros((RB, 128), f32) + b_ref[0:1, :]
            for a, m in zip(arr_refs, mat_refs):
                acc = acc + _dot(a[...], m[...])
            outs[oi][...] = acc
            oi += 1
        if oute is not None:
            outs[oi][...] = _dot(he_ref[...], w_ref[...]) + ob_ref[0:1, :]
            oi += 1
        if node is not None:
            g_out = outs[oi]
            no_out = outs[oi + 1:oi + 1 + nn_o]

            @pl.when(i <= 1)
            def _():
                nacc = _dot(ns_ref[0] + ns_ref[1], nma_ref[...])
                nacc = nacc + ((nd_ref[0] + nd_ref[1]) * nca_ref[0:1, :]
                               + nb_ref[0:1, :])
                for a, m in zip(nt_arr, nt_mat):
                    nacc = nacc + _dot(a[...], m[...])
                for a in na_arr:
                    nacc = nacc + a[...]
                g = jnp.maximum(nacc, 0.0)
                g_out[...] = g
                for o, m, bb in zip(no_out, no_mat, no_b):
                    o[...] = _dot(g, m[...]) + bb[0:1, :]

    eb = pl.BlockSpec((RB, 128), lambda i: (i, 0))
    wb = pl.BlockSpec((128, 128), lambda i: (0, 0))
    in_specs = [eb] * n_t + [wb] * n_t
    args = [a for a, _ in terms] + [_kr(m) for _, m in terms]
    out_shape = []
    out_specs = []
    if has_x:
        in_specs += [pl.BlockSpec((8, 128), lambda i: (0, 0))]
        args += [_bc8(_tile8(bias))]
        out_shape.append(jax.ShapeDtypeStruct((EP8, 128), f32))
        out_specs.append(eb)
    if oute is not None:
        he, w, ob = oute
        in_specs += [eb, pl.BlockSpec((128, 8), lambda i: (0, 0)),
                     pl.BlockSpec((8, 8), lambda i: (0, 0))]
        args += [he, _kr(w), _bc8(_tile8(ob))]
        out_shape.append(jax.ShapeDtypeStruct((EP8, 8), f32))
        out_specs.append(pl.BlockSpec((RB, 8), lambda i: (i, 0)))
    if node is not None:
        nmap = lambda i: (jnp.minimum(i, 1), 0)
        nsb = pl.BlockSpec((2, NBn, 128), lambda i: (0, jnp.minimum(i, 1), 0))
        nnb = pl.BlockSpec((NBn, 128), nmap)
        ncb = pl.BlockSpec((8, 128), lambda i: (0, 0))
        in_specs += [nsb, nsb, wb, ncb, ncb]
        args += [node['S'].reshape(2, NP8, 128), node['D'].reshape(2, NP8, 128),
                 _kr(node['Ma']), _bc8(_tile8(node['ca'])),
                 _bc8(_tile8(node['bias']))]
        in_specs += [nnb] * nn_t + [wb] * nn_t + [nnb] * nn_a
        args += [a for a, _ in node['terms']]
        args += [_kr(m) for _, m in node['terms']]
        args += list(node['adds'])
        in_specs += [pl.BlockSpec((128, 8 * m.shape[1]), lambda i: (0, 0))
                     for m, _ in node['outs']]
        in_specs += [pl.BlockSpec((8, 8 * m.shape[1]), lambda i: (0, 0))
                     for m, _ in node['outs']]
        args += [_kr(m) for m, _ in node['outs']]
        args += [_bc8(_tile8(b)) for _, b in node['outs']]
        out_shape.append(jax.ShapeDtypeStruct((NP8, 128), f32))
        out_specs.append(nnb)
        for m, _ in node['outs']:
            out_shape.append(jax.ShapeDtypeStruct((NP8, 8 * m.shape[1]), f32))
            out_specs.append(pl.BlockSpec(
                (NBn, 8 * m.shape[1]), nmap))
    res = pl.pallas_call(body, grid=(grid,), in_specs=in_specs,
                         out_specs=out_specs, out_shape=out_shape)(*args)
    return res


def _tc_prep(na2, u128, f):
    def body(na, uu, ws, wd, wu, wv, b1, ta, tb, be):
        ta[...] = _dot(na[...], ws[...])
        tb[...] = _dot(na[...], wd[...])
        u = _dot(uu[0:1, :], wu[...])
        be[...] = _dot(na[...], wv[...]) + u + b1[0:1, :]

    NB = 640
    nb = pl.BlockSpec((NB, 1024), lambda i: (i, 0))
    ob = pl.BlockSpec((NB, 128), lambda i: (i, 0))
    cb = pl.BlockSpec((8, 128), lambda i: (0, 0))
    wb = pl.BlockSpec((1024, 128), lambda i: (0, 0))
    out_shape = [jax.ShapeDtypeStruct((NP8, 128), f32)] * 3
    return pl.pallas_call(
        body, grid=(NP8 // NB,),
        in_specs=[nb, cb, wb, wb, pl.BlockSpec((128, 128), lambda i: (0, 0)),
                  wb, cb],
        out_specs=[ob, ob, ob], out_shape=out_shape)(
            na2, u128, _kr(f['ee_W1s']), _kr(f['ee_W1d']), _kr(f['en_W1u']),
            _kr(f['en_W1v']), _bc8(_tile8(f['en_b1'])))


def _tc_q1(eap_packed, W1e, b1):
    def body(ea, w, b, out):
        out[...] = _dot(ea[...], w[...]) + b[0:1, :]

    eb = pl.BlockSpec((RB, 128), lambda i: (i, 0))
    return pl.pallas_call(
        body, grid=(EP8 // RB,),
        in_specs=[eb, pl.BlockSpec((128, 128), lambda i: (0, 0)),
                  pl.BlockSpec((8, 128), lambda i: (0, 0))],
        out_specs=eb,
        out_shape=jax.ShapeDtypeStruct((EP8, 128), f32))(
            eap_packed, _kr(W1e), _bc8(_tile8(b1)))


def kernel(node_attr, edge_attr, global_attr, params, edge_index, steps):
    f = _fold(params)
    row = edge_index[0].astype(jnp.int32)
    col = edge_index[1].astype(jnp.int32)
    pad_idx = (jnp.arange(EPAD - E, dtype=jnp.int32) % 64)
    rowp = jnp.concatenate([row, pad_idx]).reshape(TOT_CH, CH)
    colp = jnp.concatenate([col, pad_idx]).reshape(TOT_CH, CH)
    dump_idx = N + (jnp.arange(EPAD - E, dtype=jnp.int32) % (NPAD - N))
    colsp = jnp.concatenate([col, dump_idx]).reshape(TOT_CH, CH)
    zeros_hbm = jnp.zeros((NPAD, 16), f32)
    ones_hbm = jnp.ones((CH, 16), f32)
    eap = jnp.concatenate([edge_attr.reshape(E // 8, 128),
                           jnp.zeros((EP8 - E // 8, 128), f32)])
    na2 = jnp.concatenate([node_attr, jnp.zeros((NPAD - N, 128), f32)]
                          ).reshape(NP8, 1024)
    u128 = jnp.broadcast_to(jnp.tile(global_attr.reshape(16), 8)[None, :],
                            (8, 128))

    Ta0, Tb0, base_en = _tc_prep(na2, u128, f)
    X0 = _tc_q1(eap, f['ee_W1e'], f['ee_b1'])

    S1, h1, D = _sc_round(Ta0.reshape(NPAD, 16), Tb0.reshape(NPAD, 16), rowp,
                         colp, colsp, X0, zeros_hbm, ones_hbm, True)
    z16 = jnp.zeros(16, f32)
    XC1, g1, TaC1, TbC1 = _tc_edge(
        [(h1, f['ce_Ae0'] + f['ce_Ae_ee'])],
        f['ce_c_v0'] + f['ce_c_en'] + f['ce_b1'],
        node=dict(S=S1, D=D, terms=[], adds=[base_en], Ma=f['en_Ma'],
                  ca=f['en_ca'], bias=z16,
                  outs=[(f['ce_Ms_v0'] + f['ce_Ms_v_en'], z16),
                        (f['ce_Md_v0'] + f['ce_Md_v_en'], z16)]))

    Sc1, hc1 = _sc_round(TaC1.reshape(NPAD, 16), TbC1.reshape(NPAD, 16),
                         rowp, colp, colsp, XC1, zeros_hbm,
                         ones_hbm, False)
    XD1, gc1, TaD1, TbD1 = _tc_edge(
        [(hc1, f['de_Ae'])], f['de_c'] + f['de_b1'],
        node=dict(S=Sc1, D=D, terms=[(g1, f['cn_Mv0'] + f['cn_Mv_en'])],
                  adds=[], Ma=f['cn_Ma'], ca=f['cn_ca'],
                  bias=f['cn_c_v0'] + f['cn_c_en'] + f['cn_b1'],
                  outs=[(f['de_Ms'], z16), (f['de_Md'], z16)]))

    Sd1, hd1 = _sc_round(TaD1.reshape(NPAD, 16), TbD1.reshape(NPAD, 16),
                         rowp, colp, colsp, XD1, zeros_hbm,
                         ones_hbm, False)
    XC2, oute1, gd1, outv1 = _tc_edge(
        [(h1, f['ce_Ae0']), (hc1, f['ce_Ae_ce'])],
        f['ce_c_v0'] + f['ce_c_cn'] + f['ce_b1'],
        oute=(hd1, f['oe_W'], f['oe_b']),
        node=dict(S=Sd1, D=D, terms=[(gc1, f['dn_Mv'])], adds=[],
                  Ma=f['dn_Ma'], ca=f['dn_ca'], bias=f['dn_cv'] + f['dn_b1'],
                  outs=[(f['ov_W'], f['ov_b'])]))
    TaC2, TbC2 = _tc_tables2(g1, gc1, f['ce_Ms_v0'], f['ce_Ms_v_cn'],
                             f['ce_Md_v0'], f['ce_Md_v_cn'])

    Sc2, hc2 = _sc_round(TaC2.reshape(NPAD, 16), TbC2.reshape(NPAD, 16),
                         rowp, colp, colsp, XC2, zeros_hbm,
                         ones_hbm, False)
    XD2, gc2, TaD2, TbD2 = _tc_edge(
        [(hc2, f['de_Ae'])], f['de_c'] + f['de_b1'],
        node=dict(S=Sc2, D=D,
                  terms=[(g1, f['cn_Mv0']), (gc1, f['cn_Mv_cn'])], adds=[],
                  Ma=f['cn_Ma'], ca=f['cn_ca'],
                  bias=f['cn_c_v0'] + f['cn_c_cn'] + f['cn_b1'],
                  outs=[(f['de_Ms'], z16), (f['de_Md'], z16)]))

    Sd2, hd2 = _sc_round(TaD2.reshape(NPAD, 16), TbD2.reshape(NPAD, 16),
                         rowp, colp, colsp, XD2, zeros_hbm,
                         ones_hbm, False)
    oute2, gd2, outv2 = _tc_edge(
        [], None, oute=(hd2, f['oe_W'], f['oe_b']),
        node=dict(S=Sd2, D=D, terms=[(gc2, f['dn_Mv'])], adds=[],
                  Ma=f['dn_Ma'], ca=f['dn_ca'], bias=f['dn_cv'] + f['dn_b1'],
                  outs=[(f['ov_W'], f['ov_b'])]))

    outs_v = jnp.stack([outv1.reshape(NPAD, 5)[:N],
                        outv2.reshape(NPAD, 5)[:N]])
    outs_e = jnp.stack([oute1.reshape(EPAD)[:E].reshape(E, 1),
                        oute2.reshape(EPAD)[:E].reshape(E, 1)])
    outs_u = jnp.zeros((2, 1, 16), f32)
    return (outs_v, outs_e, outs_u)


def _tc_tables2(ga, gb, Msa, Msb, Mda, Mdb):
    def body(a, b, m1, m2, m3, m4, ta, tb):
        ta[...] = _dot(a[...], m1[...]) + _dot(b[...], m2[...])
        tb[...] = _dot(a[...], m3[...]) + _dot(b[...], m4[...])

    NB = 640
    nb = pl.BlockSpec((NB, 128), lambda i: (i, 0))
    wb = pl.BlockSpec((128, 128), lambda i: (0, 0))
    out_shape = [jax.ShapeDtypeStruct((NP8, 128), f32)] * 2
    return pl.pallas_call(body, grid=(NP8 // NB,),
                          in_specs=[nb, nb, wb, wb, wb, wb],
                          out_specs=[nb, nb], out_shape=out_shape)(
                              ga, gb, _kr(Msa), _kr(Msb), _kr(Mda), _kr(Mdb))
